# baseline (speedup 1.0000x reference)
import functools
import jax
import jax.numpy as jnp
from jax import lax
from jax.experimental import pallas as pl
from jax.experimental.pallas import tpu as pltpu
from jax.experimental.pallas import tpu_sc as plsc

VOCAB = 100000
NADD = 1024
D = 64
BATCH = 4096
HIST = 200
B = BATCH * HIST
NC, NS = 2, 16
NW = NC * NS
BPW = B // NW
CH = 512
NCHUNK = BPW // CH
GW = 128
NSUB = CH // GW

_mesh = plsc.VectorSubcoreMesh(core_axis_name="c", subcore_axis_name="s")


@functools.partial(
    pl.kernel,
    mesh=_mesh,
    out_type=jax.ShapeDtypeStruct((B, D), jnp.float32),
    scratch_types=[
        pltpu.VMEM((2, CH), jnp.int32),
        pltpu.VMEM((2, CH, D), jnp.float32),
        pltpu.SemaphoreType.DMA,
        pltpu.SemaphoreType.DMA,
        pltpu.SemaphoreType.DMA,
        pltpu.SemaphoreType.DMA,
        pltpu.SemaphoreType.DMA,
    ],
    compiler_params=pltpu.CompilerParams(use_tc_tiling_on_sc=False),
)
def _gather_kernel(table_hbm, idx_hbm, out_hbm, idx_v, rows_v,
                   isem0, isem1, gsem, ssem0, ssem1):
    wid = lax.axis_index("s") * NC + lax.axis_index("c")
    base = wid * BPW
    isems = (isem0, isem1)
    ssems = (ssem0, ssem1)

    def idx_copy(c, b, sem):
        return pltpu.make_async_copy(
            idx_hbm.at[pl.ds(base + c * CH, CH)], idx_v.at[b], sem)

    def store_copy(c, b, sem):
        return pltpu.make_async_copy(
            rows_v.at[b], out_hbm.at[pl.ds(base + c * CH, CH)], sem)

    idx_copy(0, 0, isems[0]).start()

    def pair_body(g, _):
        for b in range(2):
            c = 2 * g + b
            @pl.when(g >= 1)
            def _wait_store():
                store_copy(c - 2, b, ssems[b]).wait()
            idx_copy(c, b, isems[b]).wait()
            for j in range(NSUB):
                pltpu.async_copy(
                    table_hbm.at[idx_v.at[b].at[pl.ds(j * GW, GW)]],
                    rows_v.at[b].at[pl.ds(j * GW, GW)],
                    gsem,
                )
            nxt = c + 1
            nxt = jnp.where(nxt == NCHUNK, 0, nxt)
            idx_copy(nxt, 1 - b, isems[1 - b]).start()
            for j in range(NSUB):
                pltpu.make_async_copy(
                    table_hbm.at[idx_v.at[b].at[pl.ds(j * GW, GW)]],
                    rows_v.at[b].at[pl.ds(j * GW, GW)],
                    gsem,
                ).wait()
            store_copy(c, b, ssems[b]).start()
        return ()

    lax.fori_loop(0, NCHUNK // 2, pair_body, ())

    store_copy(NCHUNK - 2, 0, ssems[0]).wait()
    store_copy(NCHUNK - 1, 1, ssems[1]).wait()
    idx_copy(0, 0, isems[0]).wait()


@jax.jit
def _impl(embed_frozen, weights_train, idx):
    table = jnp.concatenate((embed_frozen, weights_train), axis=0)
    idx2 = idx.reshape(B).astype(jnp.int32)
    out = _gather_kernel(table, idx2)
    return out.reshape(BATCH, HIST, D)


def kernel(embed_frozen, weights_train, idx):
    return _impl(embed_frozen, weights_train, idx)
